# baseline (speedup 1.0000x reference)
import jax
import jax.numpy as jnp
from jax.experimental import pallas as pl


def _add_kernel(x_ref, pos_ref, o_ref):
    o_ref[...] = x_ref[...] + pos_ref[...][:, None, :]


def kernel(x, pos_embedding):
    seq, batch, d = x.shape
    tile = 256
    grid = (seq // tile,)
    return pl.pallas_call(
        _add_kernel,
        grid=grid,
        in_specs=[
            pl.BlockSpec((tile, batch, d), lambda i: (i, 0, 0)),
            pl.BlockSpec((tile, d), lambda i: (i, 0)),
        ],
        out_specs=pl.BlockSpec((tile, batch, d), lambda i: (i, 0, 0)),
        out_shape=jax.ShapeDtypeStruct((seq, batch, d), x.dtype),
    )(x, pos_embedding[:seq])


# speedup vs baseline: 1.0226x; 1.0226x over previous
import jax
import jax.numpy as jnp
from jax.experimental import pallas as pl


def _add_kernel(x_ref, pos_ref, o_ref):
    o_ref[...] = x_ref[...] + pos_ref[...][:, None, :]


def kernel(x, pos_embedding):
    seq, batch, d = x.shape
    tile = 512
    grid = (seq // tile,)
    return pl.pallas_call(
        _add_kernel,
        grid=grid,
        in_specs=[
            pl.BlockSpec((tile, batch, d), lambda i: (i, 0, 0)),
            pl.BlockSpec((tile, d), lambda i: (i, 0)),
        ],
        out_specs=pl.BlockSpec((tile, batch, d), lambda i: (i, 0, 0)),
        out_shape=jax.ShapeDtypeStruct((seq, batch, d), x.dtype),
    )(x, pos_embedding[:seq])
